# baseline (speedup 1.0000x reference)
import functools

import jax
import jax.numpy as jnp
from jax import lax
from jax.experimental import pallas as pl
from jax.experimental.pallas import tpu as pltpu
from jax.experimental.pallas import tpu_sc as plsc

_NC = 2
_NS = 16
_NW = _NC * _NS
_CH = 128


_SG = 8
_C0_W = 1
_TOT_W = 1


def _sc_aggregate(table, src2, dst2, zrows, zdeg, ones, n_pad, with_deg):
  n_chunks = src2.shape[0]
  D = table.shape[1]
  n_sg_all = n_chunks // (_NS * _SG)
  n_sg0 = (n_sg_all * _C0_W + _TOT_W // 2) // _TOT_W
  n_sg1 = n_sg_all - n_sg0
  cpw0 = n_sg0 * _SG
  cpw1 = n_sg1 * _SG
  rpt = n_pad // _NS
  mesh = plsc.VectorSubcoreMesh(core_axis_name="c", subcore_axis_name="s",
                                num_cores=_NC, num_subcores=_NS)
  out_type = [jax.ShapeDtypeStruct((_NC * n_pad, D), jnp.float32)]
  scratch = [
      pltpu.VMEM((_SG, _CH), jnp.int32),
      pltpu.VMEM((_SG, _CH), jnp.int32),
      pltpu.VMEM((_CH, D), jnp.float32),
      pltpu.VMEM((_CH, D), jnp.float32),
      pltpu.SemaphoreType.DMA,
      pltpu.SemaphoreType.DMA,
      pltpu.SemaphoreType.DMA,
      pltpu.SemaphoreType.DMA,
      pltpu.SemaphoreType.DMA,
  ]
  if with_deg:
    out_type.append(jax.ShapeDtypeStruct((_NC * n_pad,), jnp.float32))
    scratch += [pltpu.VMEM((_CH,), jnp.float32),
                pltpu.VMEM_SHARED((n_pad,), jnp.float32)]
  scratch.append(pltpu.VMEM_SHARED((n_pad, D), jnp.float32))

  def body(*refs):
    if with_deg:
      (table_h, src_h, dst_h, zr_h, zd_h, on_h, acc_h, deg_h,
       src_sg, dst_sg, rows0, rows1, gsem0, gsem1, ssem0, ssem1, dsem,
       ones_v, deg_sh, acc_sh) = refs
    else:
      (table_h, src_h, dst_h, zr_h, acc_h,
       src_sg, dst_sg, rows0, rows1, gsem0, gsem1, ssem0, ssem1, dsem,
       acc_sh) = refs
    rows = (rows0, rows1)
    gsems = (gsem0, gsem1)
    ssems = (ssem0, ssem1)
    cid = lax.axis_index("c")
    sid = lax.axis_index("s")
    w = cid * _NS + sid

    r0 = sid * rpt
    pltpu.sync_copy(zr_h, acc_sh.at[pl.ds(r0, rpt)])
    if with_deg:
      pltpu.sync_copy(zd_h, deg_sh.at[pl.ds(r0, rpt)])
      pltpu.sync_copy(on_h, ones_v)
    plsc.subcore_barrier()

    crow_w = jnp.where(cid == 0, sid * cpw0, _NS * cpw0 + sid * cpw1)
    n_sg_c = jnp.where(cid == 0, n_sg0, n_sg1)

    def sg_step(s, carry):
      crow = crow_w + s * _SG
      pltpu.sync_copy(src_h.at[pl.ds(crow, _SG)], src_sg)
      gd = [pltpu.async_copy(table_h.at[src_sg.at[0]], rows0, gsem0),
            pltpu.async_copy(table_h.at[src_sg.at[1]], rows1, gsem1)]
      gd += [None] * (_SG - 2)
      pltpu.sync_copy(dst_h.at[pl.ds(crow, _SG)], dst_sg)
      dd = []
      last_sd = [None, None]
      for j in range(_SG):
        b = j % 2
        gd[j].wait()
        sd = pltpu.async_copy(rows[b], acc_sh.at[dst_sg.at[j]],
                              ssems[b], add=True)
        last_sd[b] = sd
        if with_deg:
          dd.append(pltpu.async_copy(ones_v, deg_sh.at[dst_sg.at[j]],
                                     dsem, add=True))
        if j + 2 < _SG:
          sd.wait()
          gd[j + 2] = pltpu.async_copy(table_h.at[src_sg.at[j + 2]],
                                       rows[b], gsems[b])
      last_sd[0].wait()
      last_sd[1].wait()
      for d in dd:
        d.wait()
      return carry

    lax.fori_loop(0, n_sg_c, sg_step, 0)

    plsc.subcore_barrier()
    out_r0 = cid * n_pad + sid * rpt
    pltpu.sync_copy(acc_sh.at[pl.ds(r0, rpt)], acc_h.at[pl.ds(out_r0, rpt)])
    if with_deg:
      pltpu.sync_copy(deg_sh.at[pl.ds(r0, rpt)], deg_h.at[pl.ds(out_r0, rpt)])

  kfn = pl.kernel(body, out_type=tuple(out_type), mesh=mesh,
                  scratch_types=tuple(scratch))
  if with_deg:
    return kfn(table, src2, dst2, zrows, zdeg, ones)
  return (kfn(table, src2, dst2, zrows)[0],)


def _tc_dense(h_pad, acc, deg2, Ws, Wn, b, Wtail, btail):
  n_pad, D = h_pad.shape
  H = Ws.shape[1]
  BLK = 1024
  nb = n_pad // BLK

  def body(*refs):
    if Wtail is not None:
      (h_ref, a0_ref, a1_ref, d0_ref, d1_ref, ws_ref, wn_ref, b_ref,
       wt_ref, bt_ref, o_ref) = refs
    else:
      (h_ref, a0_ref, a1_ref, d0_ref, d1_ref, ws_ref, wn_ref, b_ref,
       o_ref) = refs
    inv = 1.0 / jnp.maximum(d0_ref[...] + d1_ref[...], 1.0)
    neigh = (a0_ref[...] + a1_ref[...]) * inv
    pre = (jnp.dot(h_ref[...], ws_ref[...], preferred_element_type=jnp.float32)
           + jnp.dot(neigh, wn_ref[...], preferred_element_type=jnp.float32)
           + b_ref[...])
    hh = jnp.maximum(pre, 0.0)
    if Wtail is not None:
      o_ref[...] = (jnp.dot(hh, wt_ref[...],
                            preferred_element_type=jnp.float32) + bt_ref[...])
    else:
      o_ref[...] = hh

  in_specs = [
      pl.BlockSpec((BLK, D), lambda i: (i, 0)),
      pl.BlockSpec((BLK, D), lambda i: (i, 0)),
      pl.BlockSpec((BLK, D), lambda i: (i + nb, 0)),
      pl.BlockSpec((BLK, 1), lambda i: (i, 0)),
      pl.BlockSpec((BLK, 1), lambda i: (i + nb, 0)),
      pl.BlockSpec((D, H), lambda i: (0, 0)),
      pl.BlockSpec((D, H), lambda i: (0, 0)),
      pl.BlockSpec((1, H), lambda i: (0, 0)),
  ]
  args = [h_pad, acc, acc, deg2, deg2, Ws, Wn, b.reshape(1, H)]
  out_w = H
  if Wtail is not None:
    in_specs += [pl.BlockSpec((H, Wtail.shape[1]), lambda i: (0, 0)),
                 pl.BlockSpec((1, Wtail.shape[1]), lambda i: (0, 0))]
    args += [Wtail, btail.reshape(1, -1)]
    out_w = Wtail.shape[1]
  return pl.pallas_call(
      body,
      grid=(nb,),
      in_specs=in_specs,
      out_specs=pl.BlockSpec((BLK, out_w), lambda i: (i, 0)),
      out_shape=jax.ShapeDtypeStruct((n_pad, out_w), jnp.float32),
  )(*args)


def _sc_pair(z_full, x0p, x1p, lidx):
  H = z_full.shape[1]
  P = x0p.shape[0]
  ppc = P // _NC
  ppw = P // _NW
  mesh = plsc.VectorSubcoreMesh(core_axis_name="c", subcore_axis_name="s",
                                num_cores=_NC, num_subcores=_NS)

  @functools.partial(
      pl.kernel, mesh=mesh,
      out_type=jax.ShapeDtypeStruct((P, H), jnp.float32),
      scratch_types=(
          pltpu.VMEM((ppw,), jnp.int32),
          pltpu.VMEM((ppw,), jnp.int32),
          pltpu.VMEM((ppw, H), jnp.float32),
          pltpu.VMEM_SHARED((ppc, H), jnp.float32),
          pltpu.SemaphoreType.DMA,
      ))
  def k(z_h, x0_h, x1_h, li_h, o_h, idx_v, dst_v, rows_v, acc_sh, sem):
    cid = lax.axis_index("c")
    sid = lax.axis_index("s")
    l0 = sid * ppw
    g0 = cid * ppc + l0
    pltpu.sync_copy(li_h.at[pl.ds(l0, ppw)], dst_v)
    pltpu.sync_copy(x0_h.at[pl.ds(g0, ppw)], idx_v)
    pltpu.async_copy(z_h.at[idx_v], rows_v, sem).wait()
    pltpu.sync_copy(rows_v, acc_sh.at[pl.ds(l0, ppw)])
    pltpu.sync_copy(x1_h.at[pl.ds(g0, ppw)], idx_v)
    pltpu.async_copy(z_h.at[idx_v], rows_v, sem).wait()
    pltpu.sync_copy(rows_v, acc_sh.at[dst_v], add=True)
    plsc.subcore_barrier()
    pltpu.sync_copy(acc_sh.at[pl.ds(l0, ppw)], o_h.at[pl.ds(g0, ppw)])

  return k(z_full, x0p, x1p, lidx)


def kernel(h, edge_index, x, W_self1, W_neigh1, b1, W_self2, W_neigh2, b2,
           W_out, b_out):
  N, D = h.shape
  E = edge_index.shape[1]
  H = W_self1.shape[1]
  C = W_out.shape[1]
  P = x.shape[0]

  n_pad = (N // 1024 + 1) * 1024
  grain = _NS * _CH * _SG
  e_pad = -(-E // grain) * grain

  src = edge_index[0]
  dst = edge_index[1]
  pad_dst = N + jnp.arange(e_pad - E, dtype=jnp.int32) % (n_pad - N)
  src3 = jnp.concatenate(
      [src, jnp.zeros((e_pad - E,), jnp.int32)]).reshape(-1, _CH)
  dst3 = jnp.concatenate([dst, pad_dst]).reshape(-1, _CH)

  rpt = n_pad // _NS
  zrows = jnp.zeros((rpt, D), jnp.float32)
  zdeg = jnp.zeros((rpt,), jnp.float32)
  ones = jnp.ones((_CH,), jnp.float32)

  h_pad = jnp.concatenate([h, jnp.zeros((n_pad - N, D), jnp.float32)])

  acc1, deg = _sc_aggregate(h_pad, src3, dst3, zrows, zdeg, ones, n_pad,
                            with_deg=True)
  deg2 = deg.reshape(_NC * n_pad, 1)
  h1 = _tc_dense(h_pad, acc1, deg2, W_self1, W_neigh1, b1, None, None)

  (acc2,) = _sc_aggregate(h1, src3, dst3, zrows, zdeg, ones, n_pad,
                          with_deg=False)
  W_out_pad = jnp.pad(W_out, ((0, 0), (0, H - C)))
  b_out_pad = jnp.pad(0.5 * b_out, (0, H - C))
  z_full = _tc_dense(h1, acc2, deg2, W_self2, W_neigh2, b2,
                     W_out_pad, b_out_pad)
  lidx = jnp.arange(P // _NC, dtype=jnp.int32)
  pair_full = _sc_pair(z_full, x[:, 0], x[:, 1], lidx)
  return pair_full[:, :C]


# speedup vs baseline: 1.2973x; 1.2973x over previous
import functools

import jax
import jax.numpy as jnp
from jax import lax
from jax.experimental import pallas as pl
from jax.experimental.pallas import tpu as pltpu
from jax.experimental.pallas import tpu_sc as plsc

_NC = 2
_NS = 16
_NW = _NC * _NS
_CH = 128


_SG = 8
_C0_W = 1
_TOT_W = 2


def _sc_aggregate(table, src2, dst2, wts2, n_pad, with_deg):
  n_chunks = src2.shape[0]
  D = table.shape[1]
  n_sg_all = n_chunks // (_NS * _SG)
  n_sg0 = (n_sg_all * _C0_W + _TOT_W // 2) // _TOT_W
  n_sg1 = n_sg_all - n_sg0
  cpw0 = n_sg0 * _SG
  cpw1 = n_sg1 * _SG
  rpt = n_pad // _NS
  mesh = plsc.VectorSubcoreMesh(core_axis_name="c", subcore_axis_name="s",
                                num_cores=_NC, num_subcores=_NS)
  out_type = [jax.ShapeDtypeStruct((_NC * n_pad, D), jnp.float32)]
  scratch = [
      pltpu.VMEM((_SG, _CH), jnp.int32),
      pltpu.VMEM((_SG, _CH), jnp.int32),
      pltpu.VMEM((_CH, D), jnp.float32),
      pltpu.VMEM((_CH, D), jnp.float32),
      pltpu.SemaphoreType.DMA,
      pltpu.SemaphoreType.DMA,
      pltpu.SemaphoreType.DMA,
      pltpu.SemaphoreType.DMA,
      pltpu.SemaphoreType.DMA,
  ]
  if with_deg:
    out_type.append(jax.ShapeDtypeStruct((_NC * n_pad,), jnp.float32))
    scratch += [pltpu.VMEM((_SG, _CH), jnp.float32),
                pltpu.VMEM_SHARED((n_pad,), jnp.float32)]
  scratch.append(pltpu.VMEM_SHARED((n_pad, D), jnp.float32))

  def body(*refs):
    if with_deg:
      (table_h, src_h, dst_h, wts_h, acc_h, deg_h,
       src_sg, dst_sg, rows0, rows1, gsem0, gsem1, ssem0, ssem1, dsem,
       wts_sg, deg_sh, acc_sh) = refs
    else:
      (table_h, src_h, dst_h, acc_h,
       src_sg, dst_sg, rows0, rows1, gsem0, gsem1, ssem0, ssem1, dsem,
       acc_sh) = refs
    rows = (rows0, rows1)
    gsems = (gsem0, gsem1)
    ssems = (ssem0, ssem1)
    cid = lax.axis_index("c")
    sid = lax.axis_index("s")
    w = cid * _NS + sid

    zv = jnp.zeros((16,), jnp.float32)

    def zero_row(r, carry):
      for c in range(D // 16):
        rows0[r, pl.ds(c * 16, 16)] = zv
      return carry

    lax.fori_loop(0, _CH, zero_row, 0)
    r0 = sid * rpt
    for k in range(rpt // _CH):
      pltpu.sync_copy(rows0, acc_sh.at[pl.ds(r0 + k * _CH, _CH)])
    if with_deg:
      for k in range(rpt // _CH):
        pltpu.sync_copy(rows0.at[0], deg_sh.at[pl.ds(r0 + k * _CH, _CH)])
    plsc.subcore_barrier()

    crow_w = jnp.where(cid == 0, sid * cpw0, _NS * cpw0 + sid * cpw1)
    n_sg_c = jnp.where(cid == 0, n_sg0, n_sg1)

    def sg_step(s, carry):
      crow = crow_w + s * _SG
      pltpu.sync_copy(src_h.at[pl.ds(crow, _SG)], src_sg)
      gd = [pltpu.async_copy(table_h.at[src_sg.at[0]], rows0, gsem0),
            pltpu.async_copy(table_h.at[src_sg.at[1]], rows1, gsem1)]
      gd += [None] * (_SG - 2)
      pltpu.sync_copy(dst_h.at[pl.ds(crow, _SG)], dst_sg)
      if with_deg:
        pltpu.sync_copy(wts_h.at[pl.ds(crow, _SG)], wts_sg)
      dd = []
      last_sd = [None, None]
      for j in range(_SG):
        b = j % 2
        gd[j].wait()
        sd = pltpu.async_copy(rows[b], acc_sh.at[dst_sg.at[j]],
                              ssems[b], add=True)
        last_sd[b] = sd
        if with_deg:
          dd.append(pltpu.async_copy(wts_sg.at[j], deg_sh.at[dst_sg.at[j]],
                                     dsem, add=True))
        if j + 2 < _SG:
          sd.wait()
          gd[j + 2] = pltpu.async_copy(table_h.at[src_sg.at[j + 2]],
                                       rows[b], gsems[b])
      last_sd[0].wait()
      last_sd[1].wait()
      for d in dd:
        d.wait()
      return carry

    lax.fori_loop(0, n_sg_c, sg_step, 0)

    plsc.subcore_barrier()
    out_r0 = cid * n_pad + sid * rpt
    pltpu.sync_copy(acc_sh.at[pl.ds(r0, rpt)], acc_h.at[pl.ds(out_r0, rpt)])
    if with_deg:
      pltpu.sync_copy(deg_sh.at[pl.ds(r0, rpt)], deg_h.at[pl.ds(out_r0, rpt)])

  kfn = pl.kernel(body, out_type=tuple(out_type), mesh=mesh,
                  scratch_types=tuple(scratch))
  if with_deg:
    return kfn(table, src2, dst2, wts2)
  return (kfn(table, src2, dst2)[0],)


def _tc_dense(h_pad, acc, deg2, Ws, Wn, b, Wtail, btail, n_real=None):
  n_pad, D = h_pad.shape
  H = Ws.shape[1]
  BLK = 1024
  nb = n_pad // BLK

  def body(*refs):
    if Wtail is not None:
      (h_ref, a0_ref, a1_ref, d0_ref, d1_ref, ws_ref, wn_ref, b_ref,
       wt_ref, bt_ref, o_ref) = refs
    else:
      (h_ref, a0_ref, a1_ref, d0_ref, d1_ref, ws_ref, wn_ref, b_ref,
       o_ref) = refs
    inv = 1.0 / jnp.maximum(d0_ref[...] + d1_ref[...], 1.0)
    neigh = (a0_ref[...] + a1_ref[...]) * inv
    pre = (jnp.dot(h_ref[...], ws_ref[...], preferred_element_type=jnp.float32)
           + jnp.dot(neigh, wn_ref[...], preferred_element_type=jnp.float32)
           + b_ref[...])
    hh = jnp.maximum(pre, 0.0)
    if n_real is not None:
      i = pl.program_id(0)
      row = i * BLK + lax.broadcasted_iota(jnp.int32, (BLK, 1), 0)
      hh = jnp.where(row < n_real, hh, 0.0)
    if Wtail is not None:
      o_ref[...] = (jnp.dot(hh, wt_ref[...],
                            preferred_element_type=jnp.float32) + bt_ref[...])
    else:
      o_ref[...] = hh

  in_specs = [
      pl.BlockSpec((BLK, D), lambda i: (i, 0)),
      pl.BlockSpec((BLK, D), lambda i: (i, 0)),
      pl.BlockSpec((BLK, D), lambda i: (i + nb, 0)),
      pl.BlockSpec((BLK, 1), lambda i: (i, 0)),
      pl.BlockSpec((BLK, 1), lambda i: (i + nb, 0)),
      pl.BlockSpec((D, H), lambda i: (0, 0)),
      pl.BlockSpec((D, H), lambda i: (0, 0)),
      pl.BlockSpec((1, H), lambda i: (0, 0)),
  ]
  args = [h_pad, acc, acc, deg2, deg2, Ws, Wn, b.reshape(1, H)]
  out_w = H
  if Wtail is not None:
    in_specs += [pl.BlockSpec((H, Wtail.shape[1]), lambda i: (0, 0)),
                 pl.BlockSpec((1, Wtail.shape[1]), lambda i: (0, 0))]
    args += [Wtail, btail.reshape(1, -1)]
    out_w = Wtail.shape[1]
  return pl.pallas_call(
      body,
      grid=(nb,),
      in_specs=in_specs,
      out_specs=pl.BlockSpec((BLK, out_w), lambda i: (i, 0)),
      out_shape=jax.ShapeDtypeStruct((n_pad, out_w), jnp.float32),
  )(*args)


def _sc_pair(z_full, x0p, x1p, lidx):
  H = z_full.shape[1]
  P = x0p.shape[0]
  ppc = P // _NC
  ppw = P // _NW
  mesh = plsc.VectorSubcoreMesh(core_axis_name="c", subcore_axis_name="s",
                                num_cores=_NC, num_subcores=_NS)

  @functools.partial(
      pl.kernel, mesh=mesh,
      out_type=jax.ShapeDtypeStruct((P, H), jnp.float32),
      scratch_types=(
          pltpu.VMEM((ppw,), jnp.int32),
          pltpu.VMEM((ppw,), jnp.int32),
          pltpu.VMEM((ppw, H), jnp.float32),
          pltpu.VMEM_SHARED((ppc, H), jnp.float32),
          pltpu.SemaphoreType.DMA,
      ))
  def k(z_h, x0_h, x1_h, li_h, o_h, idx_v, dst_v, rows_v, acc_sh, sem):
    cid = lax.axis_index("c")
    sid = lax.axis_index("s")
    l0 = sid * ppw
    g0 = cid * ppc + l0
    pltpu.sync_copy(li_h.at[pl.ds(l0, ppw)], dst_v)
    pltpu.sync_copy(x0_h.at[pl.ds(g0, ppw)], idx_v)
    pltpu.async_copy(z_h.at[idx_v], rows_v, sem).wait()
    pltpu.sync_copy(rows_v, acc_sh.at[pl.ds(l0, ppw)])
    pltpu.sync_copy(x1_h.at[pl.ds(g0, ppw)], idx_v)
    pltpu.async_copy(z_h.at[idx_v], rows_v, sem).wait()
    pltpu.sync_copy(rows_v, acc_sh.at[dst_v], add=True)
    plsc.subcore_barrier()
    pltpu.sync_copy(acc_sh.at[pl.ds(l0, ppw)], o_h.at[pl.ds(g0, ppw)])

  return k(z_full, x0p, x1p, lidx)


def kernel(h, edge_index, x, W_self1, W_neigh1, b1, W_self2, W_neigh2, b2,
           W_out, b_out):
  N, D = h.shape
  E = edge_index.shape[1]
  H = W_self1.shape[1]
  C = W_out.shape[1]
  P = x.shape[0]

  n_pad = (N // 1024 + 1) * 1024
  src = edge_index[0]
  dst = edge_index[1]
  e_w = -(-E // _NW)
  epw = -(-e_w // (_CH * _SG)) * (_CH * _SG)
  e_lift = _NW * e_w
  src_w = jnp.concatenate(
      [src, jnp.zeros((e_lift - E,), jnp.int32)]).reshape(_NW, e_w)
  dst_w = jnp.concatenate(
      [dst, jnp.zeros((e_lift - E,), jnp.int32)]).reshape(_NW, e_w)
  wts_w = jnp.concatenate(
      [jnp.ones((E,), jnp.float32),
       jnp.zeros((e_lift - E,), jnp.float32)]).reshape(_NW, e_w)
  npad_w = epw - e_w
  pad_src = jnp.full((_NW, npad_w), N, jnp.int32)
  pad_dst = ((jnp.arange(_NW, dtype=jnp.int32)[:, None] * 317
              + jnp.arange(npad_w, dtype=jnp.int32)[None, :] * 41) % N)
  src3 = jnp.concatenate([src_w, pad_src], axis=1).reshape(-1, _CH)
  dst3 = jnp.concatenate([dst_w, pad_dst], axis=1).reshape(-1, _CH)
  wts3 = jnp.concatenate(
      [wts_w, jnp.zeros((_NW, npad_w), jnp.float32)], axis=1).reshape(-1, _CH)

  h_pad = jnp.concatenate([h, jnp.zeros((n_pad - N, D), jnp.float32)])

  acc1, deg = _sc_aggregate(h_pad, src3, dst3, wts3, n_pad, with_deg=True)
  deg2 = deg.reshape(_NC * n_pad, 1)
  h1 = _tc_dense(h_pad, acc1, deg2, W_self1, W_neigh1, b1, None, None,
                 n_real=N)

  (acc2,) = _sc_aggregate(h1, src3, dst3, None, n_pad, with_deg=False)
  W_out_pad = jnp.pad(W_out, ((0, 0), (0, H - C)))
  b_out_pad = jnp.pad(0.5 * b_out, (0, H - C))
  z_full = _tc_dense(h1, acc2, deg2, W_self2, W_neigh2, b2,
                     W_out_pad, b_out_pad)
  lidx = jnp.arange(P // _NC, dtype=jnp.int32)
  pair_full = _sc_pair(z_full, x[:, 0], x[:, 1], lidx)
  return pair_full[:, :C]
